# baseline (speedup 1.0000x reference)
import functools

import jax
import jax.numpy as jnp
from jax import lax
from jax.experimental import pallas as pl
from jax.experimental.pallas import tpu as pltpu
from jax.experimental.pallas import tpu_sc as plsc

D = 32
B = 16384
NC = 2
NS = 16
NW = NC * NS
BPW = B // NW
CH = 128
NCH = BPW // CH
L = 16


def _vad_body(ids_hbm, eps_hbm, mu_hbm, lv_hbm,
              lat_out, mu_out, lv_out,
              idx_v, mu_v, lv_v, eps_v, lat_v, gsem, osem):
    wid = lax.axis_index("s") * NC + lax.axis_index("c")
    base = wid * BPW

    pltpu.sync_copy(ids_hbm.at[pl.ds(wid * NCH, NCH)], idx_v)

    for j in range(NCH):
        rows = pl.ds(j * CH, CH)
        pltpu.async_copy(mu_hbm.at[idx_v.at[j]], mu_v.at[rows], gsem)
        pltpu.async_copy(lv_hbm.at[idx_v.at[j]], lv_v.at[rows], gsem)
    pltpu.sync_copy(eps_hbm.at[pl.ds(base, BPW)], eps_v)
    for j in range(NCH):
        rows = pl.ds(j * CH, CH)
        pltpu.make_async_copy(mu_hbm.at[idx_v.at[j]], mu_v.at[rows], gsem).wait()
        pltpu.make_async_copy(lv_hbm.at[idx_v.at[j]], lv_v.at[rows], gsem).wait()

    mu_cp = pltpu.async_copy(mu_v, mu_out.at[pl.ds(base, BPW)], osem)
    lv_cp = pltpu.async_copy(lv_v, lv_out.at[pl.ds(base, BPW)], osem)

    def row(r, carry):
        for h in range(D // L):
            sl = pl.ds(h * L, L)
            std = jnp.exp(lv_v[r, sl] * 0.5)
            lat_v[r, sl] = mu_v[r, sl] + eps_v[r, sl] * std
        return carry

    lax.fori_loop(0, BPW, row, 0)

    mu_cp.wait()
    lv_cp.wait()
    pltpu.sync_copy(lat_v, lat_out.at[pl.ds(base, BPW)])


@functools.partial(jax.jit, static_argnums=())
def _vad_call(ids2d, eps, weight_mu, weight_logvar):
    f32 = jnp.float32
    run = pl.kernel(
        _vad_body,
        mesh=plsc.VectorSubcoreMesh(core_axis_name="c", subcore_axis_name="s"),
        compiler_params=pltpu.CompilerParams(use_tc_tiling_on_sc=False),
        out_type=(
            jax.ShapeDtypeStruct((B, D), f32),
            jax.ShapeDtypeStruct((B, D), f32),
            jax.ShapeDtypeStruct((B, D), f32),
        ),
        scratch_types=[
            pltpu.VMEM((NCH, CH), jnp.int32),
            pltpu.VMEM((BPW, D), f32),
            pltpu.VMEM((BPW, D), f32),
            pltpu.VMEM((BPW, D), f32),
            pltpu.VMEM((BPW, D), f32),
            pltpu.SemaphoreType.DMA,
            pltpu.SemaphoreType.DMA,
        ],
    )
    return run(ids2d, eps, weight_mu, weight_logvar)


def kernel(instance_ids, eps, weight_mu, weight_logvar):
    ids2d = instance_ids.reshape(B // CH, CH)
    lat, mu, lv = _vad_call(ids2d, eps, weight_mu, weight_logvar)
    return (lat, mu, lv)
